# baseline (speedup 1.0000x reference)
import jax
import jax.numpy as jnp
import numpy as np
from jax.experimental import pallas as pl
from jax.experimental.pallas import tpu as pltpu


def _base_anchor_table():
    scales = np.asarray([32.0, 128.0, 512.0], dtype=np.float32)
    ratios = np.asarray([0.5, 1.0, 2.0], dtype=np.float32)
    h_ratios = np.sqrt(ratios)
    w_ratios = 1.0 / h_ratios
    ws = (w_ratios[:, None] * scales[None, :]).reshape(-1)
    hs = (h_ratios[:, None] * scales[None, :]).reshape(-1)
    base = np.stack([-ws, -hs, ws, hs], axis=1) / 2.0
    return np.round(base).astype(np.float32)


_BASE = _base_anchor_table()

H = W = 50
P = H * W
PPAD = 2560
A = 9
M = P * A
SROWS, SCOLS = 176, 128
K = 200
KIN = 2304
CH = 256
NH = 45
NHPAD = 128

_NEG = float("-inf")
_BIG = 1 << 30


def _mm_kernel(p_ref, w_ref, b_ref, hw_ref, hb_ref, o_ref):
    t = jnp.dot(p_ref[...], w_ref[...], preferred_element_type=jnp.float32)
    t = jnp.maximum(t + b_ref[...], 0.0)
    o_ref[...] = jnp.dot(t, hw_ref[...], preferred_element_type=jnp.float32) + hb_ref[...]


def _select_kernel(scores_ref, heads_ref, sc_ref, o_ref, s_scratch):
    scale = sc_ref[0]
    org = sc_ref[1]
    flat = jax.lax.broadcasted_iota(jnp.int32, (SROWS, SCOLS), 0) * SCOLS + \
        jax.lax.broadcasted_iota(jnp.int32, (SROWS, SCOLS), 1)
    s_scratch[...] = jnp.where(flat < M, scores_ref[...], _NEG)
    lane = jax.lax.broadcasted_iota(jnp.int32, (1, 256), 1)
    lane128 = jax.lax.broadcasted_iota(jnp.int32, (1, 128), 1)
    sub8 = jax.lax.broadcasted_iota(jnp.int32, (8, SCOLS), 0)
    lan8 = jax.lax.broadcasted_iota(jnp.int32, (8, SCOLS), 1)

    nchunks = SROWS // 8
    summary = jnp.full((1, 128), _NEG, jnp.float32)
    for j in range(nchunks):
        mj = jnp.max(s_scratch[j * 8:(j + 1) * 8, :])
        summary = jnp.where(lane128 == j, mj, summary)

    def body(k, carry):
        summary, vals, midx = carry
        vmax = jnp.max(summary)
        c = jnp.min(jnp.where(summary == vmax, lane128, _BIG))
        r0 = pl.multiple_of(c * 8, 8)
        chunk = s_scratch[pl.ds(r0, 8), :]
        m_chunk = (c * 8 + sub8) * SCOLS + lan8
        m = jnp.min(jnp.where(chunk == vmax, m_chunk, _BIG))
        chunk = jnp.where(m_chunk == m, _NEG, chunk)
        s_scratch[pl.ds(r0, 8), :] = chunk
        summary = jnp.where(lane128 == c, jnp.max(chunk), summary)
        vals = jnp.where(lane == k, vmax, vals)
        midx = jnp.where(lane == k, m, midx)
        return summary, vals, midx

    _, vals_row, midx_row = jax.lax.fori_loop(
        0, K, body,
        (summary, jnp.zeros((1, 256), jnp.float32), jnp.zeros((1, 256), jnp.int32)))

    sub = jax.lax.broadcasted_iota(jnp.int32, (256, 256), 0)
    lan = jax.lax.broadcasted_iota(jnp.int32, (256, 256), 1)
    eye = (sub == lan)
    midx_col = jnp.sum(jnp.where(eye, midx_row, 0), axis=1, keepdims=True)
    vals_col = jnp.sum(jnp.where(eye, vals_row, 0.0), axis=1, keepdims=True)

    a_col = midx_col % A
    p_col = midx_col // A
    hh = p_col // W
    ww = p_col % W

    def table(vals9):
        acc = jnp.zeros((256, 1), jnp.float32)
        for j in range(A):
            acc = jnp.where(a_col == j, jnp.float32(vals9[j]), acc)
        return acc

    b0 = table(_BASE[:, 0]); b1 = table(_BASE[:, 1])
    b2 = table(_BASE[:, 2]); b3 = table(_BASE[:, 3])
    widths = b2 - b0
    heights = b3 - b1
    cx = ww.astype(jnp.float32) * scale + b0 + 0.5 * widths
    cy = hh.astype(jnp.float32) * scale + b1 + 0.5 * heights

    lane_p = jax.lax.broadcasted_iota(jnp.int32, (256, PPAD), 1)
    oh = (lane_p == p_col).astype(jnp.float32)
    g = jnp.dot(oh, heads_ref[...], preferred_element_type=jnp.float32)

    def delta(j):
        acc = jnp.zeros((256, 1), jnp.float32)
        for aa in range(A):
            c = A + 4 * aa + j
            acc = jnp.where(a_col == aa, g[:, c:c + 1], acc)
        return acc

    dx, dy, dw, dh = delta(0), delta(1), delta(2), delta(3)
    pre_cx = dx * widths + cx
    pre_cy = dy * heights + cy
    pre_w = jnp.exp(dw) * widths
    pre_h = jnp.exp(dh) * heights
    x0 = jnp.clip(pre_cx - 0.5 * pre_w, 0.0, org)
    y0 = jnp.clip(pre_cy - 0.5 * pre_h, 0.0, org)
    x1 = jnp.clip(pre_cx + 0.5 * pre_w, 0.0, org)
    y1 = jnp.clip(pre_cy + 0.5 * pre_h, 0.0, org)
    obj = 1.0 / (1.0 + jnp.exp(-vals_col))

    subc = jax.lax.broadcasted_iota(jnp.int32, (256, 1), 0)
    cond = (x1 - x0 >= 0.001) & (y1 - y0 >= 0.001) & (subc < K)
    cond_f = cond.astype(jnp.float32)
    lt = (sub >= lan).astype(jnp.float32)
    csum = jnp.dot(lt, cond_f, preferred_element_type=jnp.float32)
    kmat = jnp.where((csum == (lan + 1).astype(jnp.float32)) & cond, 1.0, 0.0)
    total = jnp.sum(cond_f)

    jrow = jax.lax.broadcasted_iota(jnp.int32, (1, 256), 1).astype(jnp.float32)

    def compact(data_col):
        first = jnp.sum(jnp.where(subc == 0, data_col, 0.0))
        out = jnp.sum(kmat * data_col, axis=0, keepdims=True)
        return jnp.where(jrow >= total, first, out)

    rows = [compact(obj), compact(x0), compact(y0), compact(x1), compact(y1)]
    o_ref[...] = jnp.concatenate(rows + [jnp.zeros((3, 256), jnp.float32)], axis=0)


def kernel(x, conv_w, conv_b, cls_w, cls_b, bbox_w, bbox_b, image_size, org_image_size):
    org_f = jnp.asarray(org_image_size, jnp.float32)
    img_f = jnp.asarray(image_size, jnp.float32)
    sc_params = jnp.stack([org_f / img_f, org_f])

    xp = jnp.pad(jnp.transpose(x[0], (1, 2, 0)), ((1, 1), (1, 1), (0, 0)))
    patches = jnp.concatenate(
        [xp[dh:dh + H, dw:dw + W, :] for dh in range(3) for dw in range(3)],
        axis=-1).reshape(P, KIN)
    patches = jnp.pad(patches, ((0, PPAD - P), (0, 0)))
    w2 = jnp.transpose(conv_w, (2, 3, 1, 0)).reshape(KIN, CH)
    hw = jnp.concatenate([cls_w.reshape(A, CH).T, bbox_w.reshape(4 * A, CH).T], axis=1)
    hw = jnp.pad(hw, ((0, 0), (0, NHPAD - NH)))
    hb = jnp.pad(jnp.concatenate([cls_b, bbox_b]), (0, NHPAD - NH)).reshape(1, NHPAD)
    bb = conv_b.reshape(1, CH)

    blk = 256
    heads = pl.pallas_call(
        _mm_kernel,
        grid=(PPAD // blk,),
        in_specs=[
            pl.BlockSpec((blk, KIN), lambda i: (i, 0)),
            pl.BlockSpec((KIN, CH), lambda i: (0, 0)),
            pl.BlockSpec((1, CH), lambda i: (0, 0)),
            pl.BlockSpec((CH, NHPAD), lambda i: (0, 0)),
            pl.BlockSpec((1, NHPAD), lambda i: (0, 0)),
        ],
        out_specs=pl.BlockSpec((blk, NHPAD), lambda i: (i, 0)),
        out_shape=jax.ShapeDtypeStruct((PPAD, NHPAD), jnp.float32),
    )(patches, w2, bb, hw, hb)

    scores = jnp.pad(heads[:P, :A].reshape(-1), (0, SROWS * SCOLS - M)).reshape(SROWS, SCOLS)

    out = pl.pallas_call(
        _select_kernel,
        in_specs=[
            pl.BlockSpec(memory_space=pltpu.VMEM),
            pl.BlockSpec(memory_space=pltpu.VMEM),
            pl.BlockSpec(memory_space=pltpu.SMEM),
        ],
        scratch_shapes=[pltpu.VMEM((SROWS, SCOLS), jnp.float32)],
        out_shape=jax.ShapeDtypeStruct((8, 256), jnp.float32),
    )(scores, heads, sc_params)

    score = out[0, :K]
    bbox = jnp.stack([out[1, :K], out[2, :K], out[3, :K], out[4, :K]], axis=1)
    return (score, bbox)


# speedup vs baseline: 1.0019x; 1.0019x over previous
import jax
import jax.numpy as jnp
import numpy as np
from jax.experimental import pallas as pl
from jax.experimental.pallas import tpu as pltpu


def _base_anchor_table():
    scales = np.asarray([32.0, 128.0, 512.0], dtype=np.float32)
    ratios = np.asarray([0.5, 1.0, 2.0], dtype=np.float32)
    h_ratios = np.sqrt(ratios)
    w_ratios = 1.0 / h_ratios
    ws = (w_ratios[:, None] * scales[None, :]).reshape(-1)
    hs = (h_ratios[:, None] * scales[None, :]).reshape(-1)
    base = np.stack([-ws, -hs, ws, hs], axis=1) / 2.0
    return np.round(base).astype(np.float32)


_BASE = _base_anchor_table()

H = W = 50
P = H * W
PPAD = 2560
A = 9
M = P * A
SROWS, SCOLS = 176, 128
K = 200
KIN = 2304
CH = 256
NH = 45
NHPAD = 128

_NEG = float("-inf")
_BIG = 1 << 30


def _mm_kernel(p_ref, w_ref, b_ref, hw_ref, hb_ref, o_ref):
    t = jnp.dot(p_ref[...], w_ref[...], preferred_element_type=jnp.float32)
    t = jnp.maximum(t + b_ref[...], 0.0)
    o_ref[...] = jnp.dot(t, hw_ref[...], preferred_element_type=jnp.float32) + hb_ref[...]


def _select_kernel(scores_ref, heads_ref, sc_ref, o_ref, s_scratch):
    scale = sc_ref[0]
    org = sc_ref[1]
    nchunks = SROWS // 8
    flat3 = (jax.lax.broadcasted_iota(jnp.int32, (nchunks, 8, SCOLS), 0) * 8 +
             jax.lax.broadcasted_iota(jnp.int32, (nchunks, 8, SCOLS), 1)) * SCOLS + \
        jax.lax.broadcasted_iota(jnp.int32, (nchunks, 8, SCOLS), 2)
    s_scratch[...] = jnp.where(flat3 < M, scores_ref[...], _NEG)
    lane = jax.lax.broadcasted_iota(jnp.int32, (1, 256), 1)
    lane128 = jax.lax.broadcasted_iota(jnp.int32, (1, 128), 1)
    sub8 = jax.lax.broadcasted_iota(jnp.int32, (8, SCOLS), 0)
    lan8 = jax.lax.broadcasted_iota(jnp.int32, (8, SCOLS), 1)

    summary = jnp.full((1, 128), _NEG, jnp.float32)
    for j in range(nchunks):
        summary = jnp.where(lane128 == j, jnp.max(s_scratch[j]), summary)

    def body(k, carry):
        summary, vals, midx = carry
        vmax = jnp.max(summary)
        c = jnp.min(jnp.where(summary == vmax, lane128, _BIG))
        chunk = s_scratch[c]
        m_chunk = c * (8 * SCOLS) + sub8 * SCOLS + lan8
        m = jnp.min(jnp.where(chunk == vmax, m_chunk, _BIG))
        chunk = jnp.where(m_chunk == m, _NEG, chunk)
        s_scratch[c] = chunk
        summary = jnp.where(lane128 == c, jnp.max(chunk), summary)
        vals = jnp.where(lane == k, vmax, vals)
        midx = jnp.where(lane == k, m, midx)
        return summary, vals, midx

    _, vals_row, midx_row = jax.lax.fori_loop(
        0, K, body,
        (summary, jnp.zeros((1, 256), jnp.float32), jnp.zeros((1, 256), jnp.int32)))

    sub = jax.lax.broadcasted_iota(jnp.int32, (256, 256), 0)
    lan = jax.lax.broadcasted_iota(jnp.int32, (256, 256), 1)
    eye = (sub == lan)
    midx_col = jnp.sum(jnp.where(eye, midx_row, 0), axis=1, keepdims=True)
    vals_col = jnp.sum(jnp.where(eye, vals_row, 0.0), axis=1, keepdims=True)

    a_col = midx_col % A
    p_col = midx_col // A
    hh = p_col // W
    ww = p_col % W

    def table(vals9):
        acc = jnp.zeros((256, 1), jnp.float32)
        for j in range(A):
            acc = jnp.where(a_col == j, jnp.float32(vals9[j]), acc)
        return acc

    b0 = table(_BASE[:, 0]); b1 = table(_BASE[:, 1])
    b2 = table(_BASE[:, 2]); b3 = table(_BASE[:, 3])
    widths = b2 - b0
    heights = b3 - b1
    cx = ww.astype(jnp.float32) * scale + b0 + 0.5 * widths
    cy = hh.astype(jnp.float32) * scale + b1 + 0.5 * heights

    lane_p = jax.lax.broadcasted_iota(jnp.int32, (256, PPAD), 1)
    oh = (lane_p == p_col).astype(jnp.float32)
    g = jnp.dot(oh, heads_ref[...], preferred_element_type=jnp.float32)

    def delta(j):
        acc = jnp.zeros((256, 1), jnp.float32)
        for aa in range(A):
            c = A + 4 * aa + j
            acc = jnp.where(a_col == aa, g[:, c:c + 1], acc)
        return acc

    dx, dy, dw, dh = delta(0), delta(1), delta(2), delta(3)
    pre_cx = dx * widths + cx
    pre_cy = dy * heights + cy
    pre_w = jnp.exp(dw) * widths
    pre_h = jnp.exp(dh) * heights
    x0 = jnp.clip(pre_cx - 0.5 * pre_w, 0.0, org)
    y0 = jnp.clip(pre_cy - 0.5 * pre_h, 0.0, org)
    x1 = jnp.clip(pre_cx + 0.5 * pre_w, 0.0, org)
    y1 = jnp.clip(pre_cy + 0.5 * pre_h, 0.0, org)
    obj = 1.0 / (1.0 + jnp.exp(-vals_col))

    subc = jax.lax.broadcasted_iota(jnp.int32, (256, 1), 0)
    cond = (x1 - x0 >= 0.001) & (y1 - y0 >= 0.001) & (subc < K)
    cond_f = cond.astype(jnp.float32)
    lt = (sub >= lan).astype(jnp.float32)
    csum = jnp.dot(lt, cond_f, preferred_element_type=jnp.float32)
    kmat = jnp.where((csum == (lan + 1).astype(jnp.float32)) & cond, 1.0, 0.0)
    total = jnp.sum(cond_f)

    jrow = jax.lax.broadcasted_iota(jnp.int32, (1, 256), 1).astype(jnp.float32)

    def compact(data_col):
        first = jnp.sum(jnp.where(subc == 0, data_col, 0.0))
        out = jnp.sum(kmat * data_col, axis=0, keepdims=True)
        return jnp.where(jrow >= total, first, out)

    rows = [compact(obj), compact(x0), compact(y0), compact(x1), compact(y1)]
    o_ref[...] = jnp.concatenate(rows + [jnp.zeros((3, 256), jnp.float32)], axis=0)


def kernel(x, conv_w, conv_b, cls_w, cls_b, bbox_w, bbox_b, image_size, org_image_size):
    org_f = jnp.asarray(org_image_size, jnp.float32)
    img_f = jnp.asarray(image_size, jnp.float32)
    sc_params = jnp.stack([org_f / img_f, org_f])

    xp = jnp.pad(jnp.transpose(x[0], (1, 2, 0)), ((1, 1), (1, 1), (0, 0)))
    patches = jnp.concatenate(
        [xp[dh:dh + H, dw:dw + W, :] for dh in range(3) for dw in range(3)],
        axis=-1).reshape(P, KIN)
    patches = jnp.pad(patches, ((0, PPAD - P), (0, 0)))
    w2 = jnp.transpose(conv_w, (2, 3, 1, 0)).reshape(KIN, CH)
    hw = jnp.concatenate([cls_w.reshape(A, CH).T, bbox_w.reshape(4 * A, CH).T], axis=1)
    hw = jnp.pad(hw, ((0, 0), (0, NHPAD - NH)))
    hb = jnp.pad(jnp.concatenate([cls_b, bbox_b]), (0, NHPAD - NH)).reshape(1, NHPAD)
    bb = conv_b.reshape(1, CH)

    blk = 256
    heads = pl.pallas_call(
        _mm_kernel,
        grid=(PPAD // blk,),
        in_specs=[
            pl.BlockSpec((blk, KIN), lambda i: (i, 0)),
            pl.BlockSpec((KIN, CH), lambda i: (0, 0)),
            pl.BlockSpec((1, CH), lambda i: (0, 0)),
            pl.BlockSpec((CH, NHPAD), lambda i: (0, 0)),
            pl.BlockSpec((1, NHPAD), lambda i: (0, 0)),
        ],
        out_specs=pl.BlockSpec((blk, NHPAD), lambda i: (i, 0)),
        out_shape=jax.ShapeDtypeStruct((PPAD, NHPAD), jnp.float32),
    )(patches, w2, bb, hw, hb)

    scores = jnp.pad(heads[:P, :A].reshape(-1), (0, SROWS * SCOLS - M)).reshape(
        SROWS // 8, 8, SCOLS)

    out = pl.pallas_call(
        _select_kernel,
        in_specs=[
            pl.BlockSpec(memory_space=pltpu.VMEM),
            pl.BlockSpec(memory_space=pltpu.VMEM),
            pl.BlockSpec(memory_space=pltpu.SMEM),
        ],
        scratch_shapes=[pltpu.VMEM((SROWS // 8, 8, SCOLS), jnp.float32)],
        out_shape=jax.ShapeDtypeStruct((8, 256), jnp.float32),
    )(scores, heads, sc_params)

    score = out[0, :K]
    bbox = jnp.stack([out[1, :K], out[2, :K], out[3, :K], out[4, :K]], axis=1)
    return (score, bbox)


# speedup vs baseline: 1.3666x; 1.3640x over previous
import jax
import jax.numpy as jnp
import numpy as np
from jax.experimental import pallas as pl
from jax.experimental.pallas import tpu as pltpu


def _base_anchor_table():
    scales = np.asarray([32.0, 128.0, 512.0], dtype=np.float32)
    ratios = np.asarray([0.5, 1.0, 2.0], dtype=np.float32)
    h_ratios = np.sqrt(ratios)
    w_ratios = 1.0 / h_ratios
    ws = (w_ratios[:, None] * scales[None, :]).reshape(-1)
    hs = (h_ratios[:, None] * scales[None, :]).reshape(-1)
    base = np.stack([-ws, -hs, ws, hs], axis=1) / 2.0
    return np.round(base).astype(np.float32)


_BASE = _base_anchor_table()

H = W = 50
P = H * W
PPAD = 2560
A = 9
M = P * A
SROWS, SCOLS = 176, 128
K = 200
KIN = 2304
CH = 256
NH = 45
NHPAD = 128

_NEG = float("-inf")
_BIG = 1 << 30


def _mm_kernel(p_ref, w_ref, b_ref, hw_ref, hb_ref, o_ref):
    t = jnp.dot(p_ref[...], w_ref[...], preferred_element_type=jnp.float32)
    t = jnp.maximum(t + b_ref[...], 0.0)
    o_ref[...] = jnp.dot(t, hw_ref[...], preferred_element_type=jnp.float32) + hb_ref[...]


def _select_kernel(scores_ref, heads_ref, sc_ref, o_ref):
    scale = sc_ref[0]
    org = sc_ref[1]
    flat = jax.lax.broadcasted_iota(jnp.int32, (SROWS, SCOLS), 0) * SCOLS + \
        jax.lax.broadcasted_iota(jnp.int32, (SROWS, SCOLS), 1)
    lane = jax.lax.broadcasted_iota(jnp.int32, (1, 256), 1)
    s0 = jnp.where(flat < M, scores_ref[...], _NEG)

    U = 8

    def body(k, carry):
        s, vals, midx = carry
        base = k * U
        for u in range(U):
            vmax = jnp.max(s)
            m = jnp.min(jnp.where(s == vmax, flat, _BIG))
            s = jnp.where(flat == m, _NEG, s)
            vals = jnp.where(lane == base + u, vmax, vals)
            midx = jnp.where(lane == base + u, m, midx)
        return s, vals, midx

    _, vals_row, midx_row = jax.lax.fori_loop(
        0, K // U, body,
        (s0, jnp.zeros((1, 256), jnp.float32), jnp.zeros((1, 256), jnp.int32)))

    sub = jax.lax.broadcasted_iota(jnp.int32, (256, 256), 0)
    lan = jax.lax.broadcasted_iota(jnp.int32, (256, 256), 1)
    eye = (sub == lan)
    midx_col = jnp.sum(jnp.where(eye, midx_row, 0), axis=1, keepdims=True)
    vals_col = jnp.sum(jnp.where(eye, vals_row, 0.0), axis=1, keepdims=True)

    a_col = midx_col % A
    p_col = midx_col // A
    hh = p_col // W
    ww = p_col % W

    def table(vals9):
        acc = jnp.zeros((256, 1), jnp.float32)
        for j in range(A):
            acc = jnp.where(a_col == j, jnp.float32(vals9[j]), acc)
        return acc

    b0 = table(_BASE[:, 0]); b1 = table(_BASE[:, 1])
    b2 = table(_BASE[:, 2]); b3 = table(_BASE[:, 3])
    widths = b2 - b0
    heights = b3 - b1
    cx = ww.astype(jnp.float32) * scale + b0 + 0.5 * widths
    cy = hh.astype(jnp.float32) * scale + b1 + 0.5 * heights

    lane_p = jax.lax.broadcasted_iota(jnp.int32, (256, PPAD), 1)
    oh = (lane_p == p_col).astype(jnp.float32)
    g = jnp.dot(oh, heads_ref[...], preferred_element_type=jnp.float32)

    def delta(j):
        acc = jnp.zeros((256, 1), jnp.float32)
        for aa in range(A):
            c = A + 4 * aa + j
            acc = jnp.where(a_col == aa, g[:, c:c + 1], acc)
        return acc

    dx, dy, dw, dh = delta(0), delta(1), delta(2), delta(3)
    pre_cx = dx * widths + cx
    pre_cy = dy * heights + cy
    pre_w = jnp.exp(dw) * widths
    pre_h = jnp.exp(dh) * heights
    x0 = jnp.clip(pre_cx - 0.5 * pre_w, 0.0, org)
    y0 = jnp.clip(pre_cy - 0.5 * pre_h, 0.0, org)
    x1 = jnp.clip(pre_cx + 0.5 * pre_w, 0.0, org)
    y1 = jnp.clip(pre_cy + 0.5 * pre_h, 0.0, org)
    obj = 1.0 / (1.0 + jnp.exp(-vals_col))

    subc = jax.lax.broadcasted_iota(jnp.int32, (256, 1), 0)
    cond = (x1 - x0 >= 0.001) & (y1 - y0 >= 0.001) & (subc < K)
    cond_f = cond.astype(jnp.float32)
    lt = (sub >= lan).astype(jnp.float32)
    csum = jnp.dot(lt, cond_f, preferred_element_type=jnp.float32)
    kmat = jnp.where((csum == (lan + 1).astype(jnp.float32)) & cond, 1.0, 0.0)
    total = jnp.sum(cond_f)

    jrow = jax.lax.broadcasted_iota(jnp.int32, (1, 256), 1).astype(jnp.float32)

    def compact(data_col):
        first = jnp.sum(jnp.where(subc == 0, data_col, 0.0))
        out = jnp.sum(kmat * data_col, axis=0, keepdims=True)
        return jnp.where(jrow >= total, first, out)

    rows = [compact(obj), compact(x0), compact(y0), compact(x1), compact(y1)]
    o_ref[...] = jnp.concatenate(rows + [jnp.zeros((3, 256), jnp.float32)], axis=0)


def kernel(x, conv_w, conv_b, cls_w, cls_b, bbox_w, bbox_b, image_size, org_image_size):
    org_f = jnp.asarray(org_image_size, jnp.float32)
    img_f = jnp.asarray(image_size, jnp.float32)
    sc_params = jnp.stack([org_f / img_f, org_f])

    xp = jnp.pad(jnp.transpose(x[0], (1, 2, 0)), ((1, 1), (1, 1), (0, 0)))
    patches = jnp.concatenate(
        [xp[dh:dh + H, dw:dw + W, :] for dh in range(3) for dw in range(3)],
        axis=-1).reshape(P, KIN)
    patches = jnp.pad(patches, ((0, PPAD - P), (0, 0)))
    w2 = jnp.transpose(conv_w, (2, 3, 1, 0)).reshape(KIN, CH)
    hw = jnp.concatenate([cls_w.reshape(A, CH).T, bbox_w.reshape(4 * A, CH).T], axis=1)
    hw = jnp.pad(hw, ((0, 0), (0, NHPAD - NH)))
    hb = jnp.pad(jnp.concatenate([cls_b, bbox_b]), (0, NHPAD - NH)).reshape(1, NHPAD)
    bb = conv_b.reshape(1, CH)

    blk = 256
    heads = pl.pallas_call(
        _mm_kernel,
        grid=(PPAD // blk,),
        in_specs=[
            pl.BlockSpec((blk, KIN), lambda i: (i, 0)),
            pl.BlockSpec((KIN, CH), lambda i: (0, 0)),
            pl.BlockSpec((1, CH), lambda i: (0, 0)),
            pl.BlockSpec((CH, NHPAD), lambda i: (0, 0)),
            pl.BlockSpec((1, NHPAD), lambda i: (0, 0)),
        ],
        out_specs=pl.BlockSpec((blk, NHPAD), lambda i: (i, 0)),
        out_shape=jax.ShapeDtypeStruct((PPAD, NHPAD), jnp.float32),
    )(patches, w2, bb, hw, hb)

    scores = jnp.pad(heads[:P, :A].reshape(-1), (0, SROWS * SCOLS - M)).reshape(SROWS, SCOLS)

    out = pl.pallas_call(
        _select_kernel,
        in_specs=[
            pl.BlockSpec(memory_space=pltpu.VMEM),
            pl.BlockSpec(memory_space=pltpu.VMEM),
            pl.BlockSpec(memory_space=pltpu.SMEM),
        ],
        out_shape=jax.ShapeDtypeStruct((8, 256), jnp.float32),
    )(scores, heads, sc_params)

    score = out[0, :K]
    bbox = jnp.stack([out[1, :K], out[2, :K], out[3, :K], out[4, :K]], axis=1)
    return (score, bbox)


# speedup vs baseline: 2.1303x; 1.5588x over previous
import jax
import jax.numpy as jnp
import numpy as np
from jax.experimental import pallas as pl
from jax.experimental.pallas import tpu as pltpu


def _base_anchor_table():
    scales = np.asarray([32.0, 128.0, 512.0], dtype=np.float32)
    ratios = np.asarray([0.5, 1.0, 2.0], dtype=np.float32)
    h_ratios = np.sqrt(ratios)
    w_ratios = 1.0 / h_ratios
    ws = (w_ratios[:, None] * scales[None, :]).reshape(-1)
    hs = (h_ratios[:, None] * scales[None, :]).reshape(-1)
    base = np.stack([-ws, -hs, ws, hs], axis=1) / 2.0
    return np.round(base).astype(np.float32)


_BASE = _base_anchor_table()

H = W = 50
P = H * W
A = 9
M = P * A
SROWS, SCOLS = 176, 128
K = 200
KIN = 2304
CH = 256
NH = 45
NHPAD = 128

_NEG = float("-inf")
_BIG = 1 << 30


HR = 2600


def _conv_kernel(x_ref, w_ref, b_ref, hw_ref, hb_ref, o_ref, acc_ref):
    for t in range(9):
        off = (t // 3) * 52 + (t % 3)
        z = jnp.dot(x_ref[off:off + HR, :], w_ref[t * CH:(t + 1) * CH, :],
                    preferred_element_type=jnp.float32)
        if t == 0:
            acc_ref[...] = z
        else:
            acc_ref[...] += z
    act = jnp.maximum(acc_ref[...] + b_ref[...], 0.0)
    o_ref[...] = jnp.dot(act, hw_ref[...], preferred_element_type=jnp.float32) + hb_ref[...]


def _select_kernel(scores_ref, heads_ref, sc_ref, o_ref):
    scale = sc_ref[0]
    org = sc_ref[1]
    flat = jax.lax.broadcasted_iota(jnp.int32, (SROWS, SCOLS), 0) * SCOLS + \
        jax.lax.broadcasted_iota(jnp.int32, (SROWS, SCOLS), 1)
    lane = jax.lax.broadcasted_iota(jnp.int32, (1, 256), 1)
    s0 = jnp.where(flat < M, scores_ref[...], _NEG)

    U = 8

    def body(k, carry):
        s, vals, midx = carry
        base = k * U
        for u in range(U):
            vmax = jnp.max(s)
            m = jnp.min(jnp.where(s == vmax, flat, _BIG))
            s = jnp.where(flat == m, _NEG, s)
            vals = jnp.where(lane == base + u, vmax, vals)
            midx = jnp.where(lane == base + u, m, midx)
        return s, vals, midx

    _, vals_row, midx_row = jax.lax.fori_loop(
        0, K // U, body,
        (s0, jnp.zeros((1, 256), jnp.float32), jnp.zeros((1, 256), jnp.int32)))

    sub = jax.lax.broadcasted_iota(jnp.int32, (256, 256), 0)
    lan = jax.lax.broadcasted_iota(jnp.int32, (256, 256), 1)
    eye = (sub == lan)
    midx_col = jnp.sum(jnp.where(eye, midx_row, 0), axis=1, keepdims=True)
    vals_col = jnp.sum(jnp.where(eye, vals_row, 0.0), axis=1, keepdims=True)

    a_col = midx_col % A
    p_col = midx_col // A
    hh = p_col // W
    ww = p_col % W

    def table(vals9):
        acc = jnp.zeros((256, 1), jnp.float32)
        for j in range(A):
            acc = jnp.where(a_col == j, jnp.float32(vals9[j]), acc)
        return acc

    b0 = table(_BASE[:, 0]); b1 = table(_BASE[:, 1])
    b2 = table(_BASE[:, 2]); b3 = table(_BASE[:, 3])
    widths = b2 - b0
    heights = b3 - b1
    cx = ww.astype(jnp.float32) * scale + b0 + 0.5 * widths
    cy = hh.astype(jnp.float32) * scale + b1 + 0.5 * heights

    lane_p = jax.lax.broadcasted_iota(jnp.int32, (256, HR), 1)
    oh = (lane_p == (hh * 52 + ww)).astype(jnp.float32)
    g = jnp.dot(oh, heads_ref[...], preferred_element_type=jnp.float32)

    def delta(j):
        acc = jnp.zeros((256, 1), jnp.float32)
        for aa in range(A):
            c = A + 4 * aa + j
            acc = jnp.where(a_col == aa, g[:, c:c + 1], acc)
        return acc

    dx, dy, dw, dh = delta(0), delta(1), delta(2), delta(3)
    pre_cx = dx * widths + cx
    pre_cy = dy * heights + cy
    pre_w = jnp.exp(dw) * widths
    pre_h = jnp.exp(dh) * heights
    x0 = jnp.clip(pre_cx - 0.5 * pre_w, 0.0, org)
    y0 = jnp.clip(pre_cy - 0.5 * pre_h, 0.0, org)
    x1 = jnp.clip(pre_cx + 0.5 * pre_w, 0.0, org)
    y1 = jnp.clip(pre_cy + 0.5 * pre_h, 0.0, org)
    obj = 1.0 / (1.0 + jnp.exp(-vals_col))

    subc = jax.lax.broadcasted_iota(jnp.int32, (256, 1), 0)
    cond = (x1 - x0 >= 0.001) & (y1 - y0 >= 0.001) & (subc < K)
    cond_f = cond.astype(jnp.float32)
    lt = (sub >= lan).astype(jnp.float32)
    csum = jnp.dot(lt, cond_f, preferred_element_type=jnp.float32)
    kmat = jnp.where((csum == (lan + 1).astype(jnp.float32)) & cond, 1.0, 0.0)
    total = jnp.sum(cond_f)

    jrow = jax.lax.broadcasted_iota(jnp.int32, (1, 256), 1).astype(jnp.float32)

    def compact(data_col):
        first = jnp.sum(jnp.where(subc == 0, data_col, 0.0))
        out = jnp.sum(kmat * data_col, axis=0, keepdims=True)
        return jnp.where(jrow >= total, first, out)

    rows = [compact(obj), compact(x0), compact(y0), compact(x1), compact(y1)]
    o_ref[...] = jnp.concatenate(rows + [jnp.zeros((3, 256), jnp.float32)], axis=0)


def kernel(x, conv_w, conv_b, cls_w, cls_b, bbox_w, bbox_b, image_size, org_image_size):
    org_f = jnp.asarray(org_image_size, jnp.float32)
    img_f = jnp.asarray(image_size, jnp.float32)
    sc_params = jnp.stack([org_f / img_f, org_f])

    xp = jnp.pad(jnp.transpose(x[0], (1, 2, 0)), ((1, 1), (1, 1), (0, 0)))
    xflat = jnp.pad(xp.reshape(52 * 52, CH), ((0, 2712 - 52 * 52), (0, 0)))
    w2 = jnp.transpose(conv_w, (2, 3, 1, 0)).reshape(KIN, CH)
    hw = jnp.concatenate([cls_w.reshape(A, CH).T, bbox_w.reshape(4 * A, CH).T], axis=1)
    hw = jnp.pad(hw, ((0, 0), (0, NHPAD - NH)))
    hb = jnp.pad(jnp.concatenate([cls_b, bbox_b]), (0, NHPAD - NH)).reshape(1, NHPAD)
    bb = conv_b.reshape(1, CH)

    heads = pl.pallas_call(
        _conv_kernel,
        scratch_shapes=[pltpu.VMEM((HR, CH), jnp.float32)],
        out_shape=jax.ShapeDtypeStruct((HR, NHPAD), jnp.float32),
    )(xflat, w2, bb, hw, hb)

    scores = jnp.pad(heads.reshape(H, 52, NHPAD)[:, :W, :A].reshape(-1),
                     (0, SROWS * SCOLS - M)).reshape(SROWS, SCOLS)

    out = pl.pallas_call(
        _select_kernel,
        in_specs=[
            pl.BlockSpec(memory_space=pltpu.VMEM),
            pl.BlockSpec(memory_space=pltpu.VMEM),
            pl.BlockSpec(memory_space=pltpu.SMEM),
        ],
        out_shape=jax.ShapeDtypeStruct((8, 256), jnp.float32),
    )(scores, heads, sc_params)

    score = out[0, :K]
    bbox = jnp.stack([out[1, :K], out[2, :K], out[3, :K], out[4, :K]], axis=1)
    return (score, bbox)
